# baseline (speedup 1.0000x reference)
import functools

import jax
import jax.numpy as jnp
from jax import lax
from jax.experimental import pallas as pl
from jax.experimental.pallas import tpu as pltpu
from jax.experimental.pallas import tpu_sc as plsc

N = 10000
D = 128
G = 64
L = 3
BN_EPS = 1e-5

NC = 2
NS = 16
NW = NC * NS
CHUNK = 128
BLK = 1000
NBLK = N // BLK
N_ACC = 10240
ROWS_PER_TILE = N_ACC // NS
ZCOPY = 128
P_CH = 48
PH0 = (P_CH, P_CH)
PH1 = (P_CH, 14)


def _make_agg():
    mesh = plsc.VectorSubcoreMesh(core_axis_name="c", subcore_axis_name="s")
    nphase = len(PH0)
    assert len(PH1) == nphase
    for a, b in zip(PH0, PH1):
        assert a % 2 == 0 and b % 2 == 0 and 2 <= b <= a <= P_CH

    @functools.partial(
        pl.kernel,
        out_type=jax.ShapeDtypeStruct((NC, N_ACC, D), jnp.float32),
        mesh=mesh,
        scratch_types=[
            pltpu.VMEM((P_CH, 2, CHUNK), jnp.int32),
            pltpu.VMEM((CHUNK, D), jnp.float32),
            pltpu.VMEM((CHUNK, D), jnp.float32),
            pltpu.VMEM_SHARED((N_ACC, D), jnp.float32),
            pltpu.SemaphoreType.DMA,
            pltpu.SemaphoreType.DMA,
            pltpu.SemaphoreType.DMA,
        ],
    )
    def agg(ztab_hbm, idx_hbm, out_hbm, idx_v, b0, b1, acc_sh, gsem, ss0, ss1):
        c = lax.axis_index("c")
        s = lax.axis_index("s")
        wid = s * NC + c

        def zrow(r, carry):
            for k in range(D // 16):
                b0[r, pl.ds(k * 16, 16)] = jnp.zeros((16,), jnp.float32)
            return carry
        lax.fori_loop(0, CHUNK, zrow, 0)
        base = s * ROWS_PER_TILE
        for k in range(ROWS_PER_TILE // ZCOPY):
            pltpu.sync_copy(b0.at[pl.ds(0, ZCOPY)],
                            acc_sh.at[pl.ds(base + k * ZCOPY, ZCOPY)])
        plsc.subcore_barrier()

        def fire_g(j, buf):
            pltpu.async_copy(ztab_hbm.at[idx_v.at[j, 0]], buf, gsem)

        def fire_s(j, buf, sem):
            pltpu.async_copy(buf, acc_sh.at[idx_v.at[j, 1]], sem, add=True)

        def drain_g():
            pltpu.make_async_copy(ztab_hbm.at[pl.ds(0, CHUNK)], b0, gsem).wait()

        def drain_s(sem):
            pltpu.make_async_copy(b0, acc_sh.at[pl.ds(0, CHUNK)], sem).wait()

        for p in range(nphase):
            n_p = lax.select(c == 0, jnp.int32(PH0[p]), jnp.int32(PH1[p]))
            pltpu.sync_copy(idx_hbm.at[wid, p], idx_v)
            fire_g(jnp.int32(0), b0)

            def body(u, carry):
                j0 = 2 * u
                @pl.when(u >= 1)
                def _():
                    drain_s(ss1)
                fire_g(j0 + 1, b1)
                drain_g()
                fire_s(j0, b0, ss0)
                drain_s(ss0)
                @pl.when(j0 + 2 < n_p)
                def _():
                    fire_g(j0 + 2, b0)
                drain_g()
                fire_s(j0 + 1, b1, ss1)
                return carry
            lax.fori_loop(0, n_p // 2, body, 0)
            drain_s(ss1)
        plsc.subcore_barrier()

        for k in range(ROWS_PER_TILE // ZCOPY):
            off = base + k * ZCOPY
            pltpu.sync_copy(acc_sh.at[pl.ds(off, ZCOPY)],
                            b0.at[pl.ds(0, ZCOPY)])
            pltpu.sync_copy(b0.at[pl.ds(0, ZCOPY)],
                            out_hbm.at[c, pl.ds(off, ZCOPY)])

    return agg


def _mlp_body(last: bool, z_ref, parts_ref, bt_ref, w1_ref, b1_ref, w2_ref,
              b2_ref, h_ref, g_ref):
    h = z_ref[...] + parts_ref[0] + parts_ref[1]
    h = jnp.maximum(
        jnp.dot(h, w1_ref[...], preferred_element_type=jnp.float32) + b1_ref[...],
        0.0)
    h = jnp.dot(h, w2_ref[...], preferred_element_type=jnp.float32) + b2_ref[...]
    if not last:
        h = jnp.maximum(h, 0.0)
    h_ref[...] = h

    b = bt_ref[0, 0, :]
    oh_t = (lax.broadcasted_iota(jnp.int32, (G, BLK), 0) == b[None, :]
            ).astype(jnp.float32)
    gpart = jnp.dot(oh_t, h, preferred_element_type=jnp.float32)

    @pl.when(pl.program_id(0) == 0)
    def _():
        g_ref[...] = jnp.zeros_like(g_ref)
    g_ref[...] += gpart


def _make_mlp(last: bool):
    return pl.pallas_call(
        functools.partial(_mlp_body, last),
        grid=(NBLK,),
        in_specs=[
            pl.BlockSpec((BLK, D), lambda i: (i, 0)),
            pl.BlockSpec((NC, BLK, D), lambda i: (0, i, 0)),
            pl.BlockSpec((1, 1, BLK), lambda i: (i, 0, 0)),
            pl.BlockSpec((D, D), lambda i: (0, 0)),
            pl.BlockSpec((1, D), lambda i: (0, 0)),
            pl.BlockSpec((D, D), lambda i: (0, 0)),
            pl.BlockSpec((1, D), lambda i: (0, 0)),
        ],
        out_specs=[
            pl.BlockSpec((BLK, D), lambda i: (i, 0)),
            pl.BlockSpec((G, D), lambda i: (0, 0)),
        ],
        out_shape=[
            jax.ShapeDtypeStruct((N, D), jnp.float32),
            jax.ShapeDtypeStruct((G, D), jnp.float32),
        ],
    )


def kernel(x, edge_index, batch,
           W1_0, b1_0, W2_0, b2_0, gamma_0, beta_0,
           W1_1, b1_1, W2_1, b2_1, gamma_1, beta_1,
           W1_2, b1_2, W2_2, b2_2, gamma_2, beta_2):
    params = [
        (W1_0, b1_0, W2_0, b2_0, gamma_0, beta_0),
        (W1_1, b1_1, W2_1, b2_1, gamma_1, beta_1),
        (W1_2, b1_2, W2_2, b2_2, gamma_2, beta_2),
    ]
    src = edge_index[0]
    dst = edge_index[1]
    e = src.shape[0]
    n0, n1 = sum(PH0), sum(PH1)
    nphase = len(PH0)
    e_pad = NS * (n0 + n1) * CHUNK
    assert e_pad >= e and PH0 == (P_CH,) * nphase
    assert PH1[:-1] == (P_CH,) * (nphase - 1)
    src_p = jnp.concatenate([src, jnp.full((e_pad - e,), N, jnp.int32)])
    dst_p = jnp.concatenate([dst, jnp.zeros((e_pad - e,), jnp.int32)])
    split = NS * n0 * CHUNK

    def pools(flat):
        p0 = flat[:split].reshape(NS, nphase, P_CH, CHUNK)
        p1 = flat[split:].reshape(NS, n1, CHUNK)
        p1 = jnp.concatenate(
            [p1, jnp.zeros((NS, nphase * P_CH - n1, CHUNK), jnp.int32)],
            axis=1).reshape(NS, nphase, P_CH, CHUNK)
        return p0, p1
    s0, s1 = pools(src_p)
    d0, d1 = pools(dst_p)
    i0 = jnp.stack([s0, d0], axis=3)
    i1 = jnp.stack([s1, d1], axis=3)
    idx_p = jnp.stack([i0, i1], axis=1).reshape(NW, nphase, P_CH, 2, CHUNK)
    batch3 = batch.reshape(NBLK, 1, BLK)
    zero_row = jnp.zeros((1, D), jnp.float32)

    agg_fn = _make_agg()
    mlp_mid = _make_mlp(last=False)
    mlp_last = _make_mlp(last=True)

    z = x
    zs, gs = [], []
    for l in range(L):
        W1, b1, W2, b2, gamma, beta = params[l]
        scale = gamma / jnp.sqrt(1.0 + BN_EPS)
        w2f = W2 * scale[None, :]
        b2f = (b2 * scale + beta).reshape(1, D)
        b1r = b1.reshape(1, D)

        ztab = jnp.concatenate([z, zero_row], axis=0)
        parts = agg_fn(ztab, idx_p)
        mlp = mlp_last if l == L - 1 else mlp_mid
        h, g = mlp(z, parts, batch3, W1, b1r, w2f, b2f)
        zs.append(h)
        gs.append(g)
        z = h

    return (jnp.concatenate(zs, axis=1), jnp.concatenate(gs, axis=1))
